# baseline (speedup 1.0000x reference)
import functools

import jax
import jax.numpy as jnp
from jax import lax
from jax.experimental import pallas as pl
from jax.experimental.pallas import tpu as pltpu
from jax.experimental.pallas import tpu_sc as plsc

V = 100000
D = 128
K = 5

NW = 16
PER_W = 6272
PAD_V = NW * PER_W
GROUPS = PER_W // 16
GM_PAD = 400

BLK = 14336
CHUNKS = BLK // D
GRID = PAD_V // BLK
ROWS_OUT = PAD_V // D

_NEG = float("-inf")
_IMAX = 2**31 - 1


def _dss_body(psi_ref, tab_ref, d_ref, ss_ref):
    psi = psi_ref[:, :]
    nrm = jnp.sqrt(jnp.sum(psi * psi)) + jnp.float32(1e-9)
    pn = psi / nrm
    one = jnp.ones((1, D), jnp.bfloat16)
    tdims = (((1,), (1,)), ((), ()))
    f32 = jnp.float32
    ds, sss = [], []
    for c in range(CHUNKS):
        chunk = tab_ref[c * D:(c + 1) * D, :]
        dc = jnp.sum(chunk * pn, axis=1, keepdims=True)
        ds.append(lax.transpose(dc, (1, 0)))
        sq = chunk * chunk
        s_hi = sq.astype(jnp.bfloat16)
        s_lo = (sq - s_hi.astype(f32)).astype(jnp.bfloat16)
        sss.append(lax.dot_general(one, s_hi, tdims, preferred_element_type=f32)
                   + lax.dot_general(one, s_lo, tdims,
                                     preferred_element_type=f32))
    d_ref[:, :] = jnp.concatenate(ds, axis=0)
    ss_ref[:, :] = jnp.concatenate(sss, axis=0)


def _dss(psi2d, table):
    return pl.pallas_call(
        _dss_body,
        grid=(GRID,),
        in_specs=[
            pl.BlockSpec((1, D), lambda i: (0, 0)),
            pl.BlockSpec((BLK, D), lambda i: (i, 0)),
        ],
        out_specs=[
            pl.BlockSpec((CHUNKS, D), lambda i: (i, 0)),
            pl.BlockSpec((CHUNKS, D), lambda i: (i, 0)),
        ],
        out_shape=[
            jax.ShapeDtypeStruct((ROWS_OUT, D), jnp.float32),
            jax.ShapeDtypeStruct((ROWS_OUT, D), jnp.float32),
        ],
    )(psi2d, table)


@functools.partial(
    pl.kernel,
    mesh=plsc.VectorSubcoreMesh(core_axis_name="c", subcore_axis_name="s",
                                num_cores=1),
    out_type=[
        jax.ShapeDtypeStruct((NW, 16), jnp.float32),
        jax.ShapeDtypeStruct((NW, 16), jnp.int32),
        jax.ShapeDtypeStruct((16,), jnp.float32),
        jax.ShapeDtypeStruct((16,), jnp.int32),
    ],
    scratch_types=[
        pltpu.VMEM((PER_W,), jnp.float32),
        pltpu.VMEM((PER_W,), jnp.float32),
        pltpu.VMEM((PER_W,), jnp.float32),
        pltpu.VMEM((GM_PAD,), jnp.float32),
        pltpu.VMEM((16,), jnp.float32),
        pltpu.VMEM((16,), jnp.int32),
        pltpu.VMEM((NW, 16), jnp.float32),
        pltpu.VMEM((NW, 16), jnp.int32),
    ],
    compiler_params=pltpu.CompilerParams(needs_layout_passes=False),
)
def _topk_sc(d_hbm, ss_hbm, candv_hbm, candi_hbm, outv_hbm, outi_hbm,
             bufd, bufss, sims, gm, candv, candi, mcv, mci):
    wid = lax.axis_index("s")
    base = wid * PER_W
    pltpu.sync_copy(d_hbm.at[pl.ds(base, PER_W)], bufd)
    pltpu.sync_copy(ss_hbm.at[pl.ds(base, PER_W)], bufss)
    iota = lax.iota(jnp.int32, 16)
    lane0 = iota == 0
    vlim = jnp.int32(V) - base

    def simbody(g, _):
        d = bufd[pl.ds(g * 16, 16)]
        ss = bufss[pl.ds(g * 16, 16)]
        y = plsc.bitcast(
            jnp.int32(0x5F3759DF) - (plsc.bitcast(ss, jnp.int32) >> 1),
            jnp.float32)
        half_ss = ss * jnp.float32(0.5)
        for _ in range(3):
            y = y * (jnp.float32(1.5) - half_ss * y * y)
        sim = d / (ss * y + jnp.float32(1e-9))
        valid = (iota + g * 16) < vlim
        sim = jnp.where(valid, sim, _NEG)
        sims[pl.ds(g * 16, 16)] = sim
        gmx = jnp.max(sim)
        plsc.store_scatter(gm, [jnp.full((16,), g, jnp.int32)],
                           jnp.full((16,), gmx, jnp.float32), mask=lane0)
        return 0

    lax.fori_loop(0, GROUPS, simbody, 0, unroll=8)
    tail = gm[pl.ds(GM_PAD - 16, 16)]
    gm[pl.ds(GM_PAD - 16, 16)] = jnp.where(iota < GROUPS % 16, tail, _NEG)

    cv = jnp.full((16,), _NEG, jnp.float32)
    ci = jnp.zeros((16,), jnp.int32)
    for k in range(K):
        gbv = jnp.full((16,), _NEG, jnp.float32)
        gbi = jnp.zeros((16,), jnp.int32)
        for t in range(GM_PAD // 16):
            v = gm[pl.ds(t * 16, 16)]
            m = v > gbv
            gbv = jnp.where(m, v, gbv)
            gbi = jnp.where(m, iota + t * 16, gbi)
        wv = jnp.max(gbv)
        gstar = jnp.min(jnp.where(gbv == wv, gbi, _IMAX))
        v = sims[pl.ds(gstar * 16, 16)]
        wi = jnp.min(jnp.where(v == wv, gstar * 16 + iota, _IMAX))
        cv = jnp.where(iota == k, wv, cv)
        ci = jnp.where(iota == k, wi + base, ci)
        if k < K - 1:
            plsc.store_scatter(sims, [jnp.full((16,), wi, jnp.int32)],
                               jnp.full((16,), _NEG, jnp.float32), mask=lane0)
            v2 = sims[pl.ds(gstar * 16, 16)]
            plsc.store_scatter(gm, [jnp.full((16,), gstar, jnp.int32)],
                               jnp.full((16,), jnp.max(v2), jnp.float32),
                               mask=lane0)
    candv[...] = cv
    candi[...] = ci
    pltpu.sync_copy(candv, candv_hbm.at[wid])
    pltpu.sync_copy(candi, candi_hbm.at[wid])
    plsc.subcore_barrier()

    @pl.when(wid == 0)
    def _():
        pltpu.sync_copy(candv_hbm, mcv)
        pltpu.sync_copy(candi_hbm, mci)
        resv = jnp.full((16,), _NEG, jnp.float32)
        resi = jnp.zeros((16,), jnp.int32)
        wvs = []
        wis = []
        for k in range(K):
            bv = jnp.full((16,), _NEG, jnp.float32)
            bi = jnp.zeros((16,), jnp.int32)
            for g in range(NW):
                v = mcv[g, :]
                idx = mci[g, :]
                for j in range(k):
                    v = jnp.where(idx == wis[j], _NEG, v)
                m = v > bv
                bv = jnp.where(m, v, bv)
                bi = jnp.where(m, idx, bi)
            wv = jnp.max(bv)
            wi = jnp.min(jnp.where(bv == wv, bi, _IMAX))
            wvs.append(wv)
            wis.append(wi)
            resv = jnp.where(iota == k, wv, resv)
            resi = jnp.where(iota == k, wi, resi)
        candv[...] = resv
        candi[...] = resi
        pltpu.sync_copy(candv, outv_hbm)
        pltpu.sync_copy(candi, outi_hbm)


def kernel(psi_final_abstract, quantum_representations, top_k):
    del top_k
    psi2d = psi_final_abstract.reshape(1, D)
    d, ss = _dss(psi2d, quantum_representations)
    _, _, tv, ti = _topk_sc(d.reshape(PAD_V), ss.reshape(PAD_V))
    return tv[:K], ti[:K]


# speedup vs baseline: 1.1994x; 1.1994x over previous
import functools

import jax
import jax.numpy as jnp
from jax import lax
from jax.experimental import pallas as pl
from jax.experimental.pallas import tpu as pltpu
from jax.experimental.pallas import tpu_sc as plsc

V = 100000
D = 128
K = 5

NW = 16
PER_W = 6272
PAD_V = NW * PER_W
GROUPS = PER_W // 16
GM_PAD = 400

BLK = 14336
CHUNKS = BLK // D
GRID = PAD_V // BLK
ROWS_OUT = PAD_V // D

_NEG = float("-inf")
_IMAX = 2**31 - 1


def _dss_body(psi_ref, tab_ref, d_ref, ss_ref):
    psi = psi_ref[:, :]
    nrm = jnp.sqrt(jnp.sum(psi * psi)) + jnp.float32(1e-9)
    pn = psi / nrm
    one = jnp.ones((1, D), jnp.bfloat16)
    tdims = (((1,), (1,)), ((), ()))
    f32 = jnp.float32
    ds, sss = [], []
    for c in range(CHUNKS):
        chunk = tab_ref[c * D:(c + 1) * D, :]
        dc = jnp.sum(chunk * pn, axis=1, keepdims=True)
        ds.append(lax.transpose(dc, (1, 0)))
        sq = chunk * chunk
        s_hi = sq.astype(jnp.bfloat16)
        s_lo = (sq - s_hi.astype(f32)).astype(jnp.bfloat16)
        sss.append(lax.dot_general(one, s_hi, tdims, preferred_element_type=f32)
                   + lax.dot_general(one, s_lo, tdims,
                                     preferred_element_type=f32))
    d_ref[:, :] = jnp.concatenate(ds, axis=0)
    ss_ref[:, :] = jnp.concatenate(sss, axis=0)


def _dss(psi2d, table):
    return pl.pallas_call(
        _dss_body,
        grid=(GRID,),
        in_specs=[
            pl.BlockSpec((1, D), lambda i: (0, 0)),
            pl.BlockSpec((BLK, D), lambda i: (i, 0)),
        ],
        out_specs=[
            pl.BlockSpec((CHUNKS, D), lambda i: (i, 0)),
            pl.BlockSpec((CHUNKS, D), lambda i: (i, 0)),
        ],
        out_shape=[
            jax.ShapeDtypeStruct((ROWS_OUT, D), jnp.float32),
            jax.ShapeDtypeStruct((ROWS_OUT, D), jnp.float32),
        ],
    )(psi2d, table)


@functools.partial(
    pl.kernel,
    mesh=plsc.VectorSubcoreMesh(core_axis_name="c", subcore_axis_name="s",
                                num_cores=1),
    out_type=[
        jax.ShapeDtypeStruct((NW, 16), jnp.float32),
        jax.ShapeDtypeStruct((NW, 16), jnp.int32),
        jax.ShapeDtypeStruct((16,), jnp.float32),
        jax.ShapeDtypeStruct((16,), jnp.int32),
    ],
    scratch_types=[
        pltpu.VMEM((PER_W,), jnp.float32),
        pltpu.VMEM((PER_W,), jnp.float32),
        pltpu.VMEM((PER_W,), jnp.float32),
        pltpu.VMEM((GM_PAD,), jnp.float32),
        pltpu.VMEM((16,), jnp.float32),
        pltpu.VMEM((16,), jnp.int32),
        pltpu.VMEM((NW, 16), jnp.float32),
        pltpu.VMEM((NW, 16), jnp.int32),
    ],
    compiler_params=pltpu.CompilerParams(needs_layout_passes=False),
)
def _topk_sc(d_hbm, ss_hbm, candv_hbm, candi_hbm, outv_hbm, outi_hbm,
             bufd, bufss, sims, gm, candv, candi, mcv, mci):
    wid = lax.axis_index("s")
    base = wid * PER_W
    pltpu.sync_copy(d_hbm.at[pl.ds(base, PER_W)], bufd)
    pltpu.sync_copy(ss_hbm.at[pl.ds(base, PER_W)], bufss)
    iota = lax.iota(jnp.int32, 16)
    lane0 = iota == 0
    vlim = jnp.int32(V) - base

    @plsc.parallel_loop(0, GROUPS, unroll=8)
    def simbody(g):
        d = bufd[pl.ds(g * 16, 16)]
        ss = bufss[pl.ds(g * 16, 16)]
        y = plsc.bitcast(
            jnp.int32(0x5F3759DF) - (plsc.bitcast(ss, jnp.int32) >> 1),
            jnp.float32)
        half_ss = ss * jnp.float32(0.5)
        for _ in range(3):
            y = y * (jnp.float32(1.5) - half_ss * y * y)
        sim = d / (ss * y + jnp.float32(1e-9))
        valid = (iota + g * 16) < vlim
        sim = jnp.where(valid, sim, _NEG)
        sims[pl.ds(g * 16, 16)] = sim
        gmx = jnp.max(sim)
        plsc.store_scatter(gm, [jnp.full((16,), g, jnp.int32)],
                           jnp.full((16,), gmx, jnp.float32), mask=lane0)

    tail = gm[pl.ds(GM_PAD - 16, 16)]
    gm[pl.ds(GM_PAD - 16, 16)] = jnp.where(iota < GROUPS % 16, tail, _NEG)

    cv = jnp.full((16,), _NEG, jnp.float32)
    ci = jnp.zeros((16,), jnp.int32)
    for k in range(K):
        gbv = jnp.full((16,), _NEG, jnp.float32)
        gbi = jnp.zeros((16,), jnp.int32)
        for t in range(GM_PAD // 16):
            v = gm[pl.ds(t * 16, 16)]
            m = v > gbv
            gbv = jnp.where(m, v, gbv)
            gbi = jnp.where(m, iota + t * 16, gbi)
        wv = jnp.max(gbv)
        gstar = jnp.min(jnp.where(gbv == wv, gbi, _IMAX))
        v = sims[pl.ds(gstar * 16, 16)]
        wi = jnp.min(jnp.where(v == wv, gstar * 16 + iota, _IMAX))
        cv = jnp.where(iota == k, wv, cv)
        ci = jnp.where(iota == k, wi + base, ci)
        if k < K - 1:
            plsc.store_scatter(sims, [jnp.full((16,), wi, jnp.int32)],
                               jnp.full((16,), _NEG, jnp.float32), mask=lane0)
            v2 = sims[pl.ds(gstar * 16, 16)]
            plsc.store_scatter(gm, [jnp.full((16,), gstar, jnp.int32)],
                               jnp.full((16,), jnp.max(v2), jnp.float32),
                               mask=lane0)
    candv[...] = cv
    candi[...] = ci
    pltpu.sync_copy(candv, candv_hbm.at[wid])
    pltpu.sync_copy(candi, candi_hbm.at[wid])
    plsc.subcore_barrier()

    @pl.when(wid == 0)
    def _():
        pltpu.sync_copy(candv_hbm, mcv)
        pltpu.sync_copy(candi_hbm, mci)
        resv = jnp.full((16,), _NEG, jnp.float32)
        resi = jnp.zeros((16,), jnp.int32)
        wvs = []
        wis = []
        for k in range(K):
            bv = jnp.full((16,), _NEG, jnp.float32)
            bi = jnp.zeros((16,), jnp.int32)
            for g in range(NW):
                v = mcv[g, :]
                idx = mci[g, :]
                for j in range(k):
                    v = jnp.where(idx == wis[j], _NEG, v)
                m = v > bv
                bv = jnp.where(m, v, bv)
                bi = jnp.where(m, idx, bi)
            wv = jnp.max(bv)
            wi = jnp.min(jnp.where(bv == wv, bi, _IMAX))
            wvs.append(wv)
            wis.append(wi)
            resv = jnp.where(iota == k, wv, resv)
            resi = jnp.where(iota == k, wi, resi)
        candv[...] = resv
        candi[...] = resi
        pltpu.sync_copy(candv, outv_hbm)
        pltpu.sync_copy(candi, outi_hbm)


def kernel(psi_final_abstract, quantum_representations, top_k):
    del top_k
    psi2d = psi_final_abstract.reshape(1, D)
    d, ss = _dss(psi2d, quantum_representations)
    _, _, tv, ti = _topk_sc(d.reshape(PAD_V), ss.reshape(PAD_V))
    return tv[:K], ti[:K]
